# baseline (speedup 1.0000x reference)
import jax
import jax.numpy as jnp
from jax import lax
from jax.experimental import pallas as pl
from jax.experimental.pallas import tpu as pltpu
from jax.experimental.pallas import tpu_sc as plsc

BATCH = 4096
SEQ = 200
EMBED = 64
VOCAB2 = 1000002

NC = 2
NS = 16
NW = NC * NS
BBLK = BATCH // NW
NB = 2

_mesh = plsc.VectorSubcoreMesh(core_axis_name="c", subcore_axis_name="s")

NBLK_FULL = (VOCAB2 // 2) // 64
TAIL_V = NBLK_FULL * 128
PAIR_PAD = NBLK_FULL * 64 + 40


def _k1_body(tT, tailp, tab2, sA, sB, pA, pB, tailv, rA, rB, wvA, wvB):
    slabs = (sA, sB)
    slabPs = (pA, pB)
    rsems = (rA, rB)
    wsems = (wvA, wvB)
    wid = lax.axis_index("s") * NC + lax.axis_index("c")
    iota16 = lax.iota(jnp.int32, 16)

    def transpose_blk(j, nq):
        def body(q, c):
            lo = jnp.full((16,), 0, jnp.int32) + 2 * q
            hi = lo + 1
            for k in range(8):
                rows = iota16 + 16 * (k % 4)
                cols = lo if k < 4 else hi
                val = plsc.load_gather(slabs[j], [rows, cols])
                slabPs[j][q, pl.ds(16 * k, 16)] = val
            return c

        lax.fori_loop(0, nq, body, 0)

    for j in range(2):
        pltpu.async_copy(
            tT.at[:, pl.ds((32 * j + wid) * 128, 128)], slabs[j], rsems[j]
        )

    def step(tt, c):
        for j in range(2):
            t = 2 * tt + j
            blk = 32 * t + wid

            @pl.when(blk < NBLK_FULL)
            def _():
                pltpu.make_async_copy(
                    tT.at[:, pl.ds(blk * 128, 128)], slabs[j], rsems[j]
                ).wait()

                @pl.when(t >= 2)
                def _():
                    pltpu.make_async_copy(
                        slabPs[j], tab2.at[pl.ds((blk - 64) * 64, 64)], wsems[j]
                    ).wait()

                transpose_blk(j, 64)
                pltpu.async_copy(
                    slabPs[j], tab2.at[pl.ds(blk * 64, 64)], wsems[j]
                )

                @pl.when(blk + 64 < NBLK_FULL)
                def _():
                    pltpu.async_copy(
                        tT.at[:, pl.ds((blk + 64) * 128, 128)], slabs[j], rsems[j]
                    )

        return c

    lax.fori_loop(0, 123, step, 0)

    for j in range(2):
        pltpu.make_async_copy(slabPs[j], tab2.at[pl.ds(0, 64)], wsems[j]).wait()

    @pl.when(wid == NW - 1)
    def _():
        pltpu.sync_copy(tailp, tailv)
        pltpu.sync_copy(tailv, tab2.at[pl.ds(NBLK_FULL * 64, 40)])


def _body(wT, tab2, out, idxv, idx2v, bA, bB, tA, tB, gA, gB, wA, wB):
    bufs = (bA, bB)
    bufTs = (tA, tB)
    gsems = (gA, gB)
    wsems = (wA, wB)
    wid = lax.axis_index("s") * NC + lax.axis_index("c")
    b0 = wid * BBLK

    pltpu.sync_copy(wT.at[:, pl.ds(b0, BBLK)], idxv)

    def prep(r, c):
        for k in range(BBLK // 16):
            v = idxv[r, pl.ds(16 * k, 16)]
            idx2v[r, pl.ds(16 * k, 16)] = lax.shift_right_logical(v, 1)
        return c

    lax.fori_loop(0, SEQ, prep, 0)

    iota16 = lax.iota(jnp.int32, 16)

    def transpose_group(s, j):
        pcs = tuple(
            lax.shift_left(
                lax.bitwise_and(idxv[s, pl.ds(16 * k, 16)], 1), 6
            )
            for k in range(BBLK // 16)
        )

        def trans(d, pcs):
            dd = jnp.full((16,), 0, jnp.int32) + d
            for k in range(BBLK // 16):
                rows = iota16 + (16 * k)
                cols = pcs[k] + dd
                val = plsc.load_gather(bufs[j], [rows, cols])
                bufTs[j][d, pl.ds(16 * k, 16)] = val
            return pcs

        lax.fori_loop(0, EMBED, trans, pcs)

    for j in range(NB):
        pltpu.async_copy(tab2.at[idx2v.at[j]], bufs[j], gsems[j])

    def step(t, carry):
        for j in range(NB):
            s = t * NB + j
            pltpu.make_async_copy(tab2.at[idx2v.at[s]], bufs[j], gsems[j]).wait()

            @pl.when(s >= NB)
            def _():
                pltpu.make_async_copy(
                    bufTs[j], out.at[s - NB, :, pl.ds(b0, BBLK)], wsems[j]
                ).wait()

            transpose_group(s, j)
            pltpu.async_copy(bufTs[j], out.at[s, :, pl.ds(b0, BBLK)], wsems[j])

            @pl.when(s + NB < SEQ)
            def _():
                pltpu.async_copy(tab2.at[idx2v.at[s + NB]], bufs[j], gsems[j])
        return carry

    lax.fori_loop(0, SEQ // NB, step, 0)

    for j in range(NB):
        s_last = SEQ - NB + j
        pltpu.make_async_copy(
            bufTs[j], out.at[s_last, :, pl.ds(b0, BBLK)], wsems[j]
        ).wait()


def _pairify(tT, tailp):
    run = pl.kernel(
        _k1_body,
        out_type=jax.ShapeDtypeStruct((PAIR_PAD, 128), jnp.float32),
        mesh=_mesh,
        scratch_types=[pltpu.VMEM((EMBED, 128), jnp.float32) for _ in range(4)]
        + [pltpu.VMEM((40, 128), jnp.float32)]
        + [pltpu.SemaphoreType.DMA] * 4,
        compiler_params=pltpu.CompilerParams(needs_layout_passes=False),
    )
    return run(tT, tailp)


@jax.jit
def _embed(wT, tab2):
    run = pl.kernel(
        _body,
        out_type=jax.ShapeDtypeStruct((SEQ, EMBED, BATCH), jnp.float32),
        mesh=_mesh,
        scratch_types=[
            pltpu.VMEM((SEQ, BBLK), jnp.int32),
            pltpu.VMEM((SEQ, BBLK), jnp.int32),
        ]
        + [pltpu.VMEM((BBLK, 128), jnp.float32) for _ in range(NB)]
        + [pltpu.VMEM((EMBED, BBLK), jnp.float32) for _ in range(NB)]
        + [pltpu.SemaphoreType.DMA] * (2 * NB),
        compiler_params=pltpu.CompilerParams(needs_layout_passes=False),
    )
    return run(wT, tab2)


def kernel(word_batch, table):
    wT = word_batch.astype(jnp.int32).T
    tableT = table.T
    tail = table[TAIL_V : VOCAB2 - 2].reshape(32, 128)
    tailp = jnp.concatenate([tail, jnp.zeros((8, 128), jnp.float32)], axis=0)
    tab2 = _pairify(tableT, tailp)
    outT = _embed(wT, tab2)
    return jnp.transpose(outT, (2, 0, 1))


# speedup vs baseline: 1.8770x; 1.8770x over previous
import jax
import jax.numpy as jnp
from jax import lax
from jax.experimental import pallas as pl
from jax.experimental.pallas import tpu as pltpu
from jax.experimental.pallas import tpu_sc as plsc

BATCH = 4096
SEQ = 200
EMBED = 64
VOCAB2 = 1000002

NC = 2
NS = 16
NW = NC * NS
BBLK = BATCH // NW
NB = 2

_mesh = plsc.VectorSubcoreMesh(core_axis_name="c", subcore_axis_name="s")

NBLK_FULL = (VOCAB2 // 2) // 64
TAIL_V = NBLK_FULL * 128
PAIR_PAD = NBLK_FULL * 64 + 40


def _k1_body(tT, tailp, tab2, sA, sB, pA, pB, tailv, rA, rB, wvA, wvB):
    slabs = (sA, sB)
    slabPs = (pA, pB)
    rsems = (rA, rB)
    wsems = (wvA, wvB)
    wid = lax.axis_index("s") * NC + lax.axis_index("c")
    iota16 = lax.iota(jnp.int32, 16)

    def transpose_blk(j, nq):
        @plsc.parallel_loop(0, nq, unroll=8)
        def body(q):
            lo = jnp.full((16,), 0, jnp.int32) + 2 * q
            hi = lo + 1
            for k in range(8):
                rows = iota16 + 16 * (k % 4)
                cols = lo if k < 4 else hi
                val = plsc.load_gather(slabs[j], [rows, cols])
                slabPs[j][q, pl.ds(16 * k, 16)] = val

    for j in range(2):
        pltpu.async_copy(
            tT.at[:, pl.ds((32 * j + wid) * 128, 128)], slabs[j], rsems[j]
        )

    def step(tt, c):
        for j in range(2):
            t = 2 * tt + j
            blk = 32 * t + wid

            @pl.when(blk < NBLK_FULL)
            def _():
                pltpu.make_async_copy(
                    tT.at[:, pl.ds(blk * 128, 128)], slabs[j], rsems[j]
                ).wait()

                @pl.when(t >= 2)
                def _():
                    pltpu.make_async_copy(
                        slabPs[j], tab2.at[pl.ds((blk - 64) * 64, 64)], wsems[j]
                    ).wait()

                transpose_blk(j, 64)
                pltpu.async_copy(
                    slabPs[j], tab2.at[pl.ds(blk * 64, 64)], wsems[j]
                )

                @pl.when(blk + 64 < NBLK_FULL)
                def _():
                    pltpu.async_copy(
                        tT.at[:, pl.ds((blk + 64) * 128, 128)], slabs[j], rsems[j]
                    )

        return c

    lax.fori_loop(0, 123, step, 0)

    for j in range(2):
        pltpu.make_async_copy(slabPs[j], tab2.at[pl.ds(0, 64)], wsems[j]).wait()

    @pl.when(wid == NW - 1)
    def _():
        pltpu.sync_copy(tailp, tailv)
        pltpu.sync_copy(tailv, tab2.at[pl.ds(NBLK_FULL * 64, 40)])


def _body(wT, tab2, out, idxv, idx2v, bA, bB, tA, tB, gA, gB, wA, wB):
    bufs = (bA, bB)
    bufTs = (tA, tB)
    gsems = (gA, gB)
    wsems = (wA, wB)
    wid = lax.axis_index("s") * NC + lax.axis_index("c")
    b0 = wid * BBLK

    pltpu.sync_copy(wT.at[:, pl.ds(b0, BBLK)], idxv)

    @plsc.parallel_loop(0, SEQ, unroll=4)
    def prep(r):
        for k in range(BBLK // 16):
            v = idxv[r, pl.ds(16 * k, 16)]
            idx2v[r, pl.ds(16 * k, 16)] = lax.shift_right_logical(v, 1)

    iota16 = lax.iota(jnp.int32, 16)

    def transpose_group(s, j):
        pcs = tuple(
            lax.shift_left(
                lax.bitwise_and(idxv[s, pl.ds(16 * k, 16)], 1), 6
            )
            for k in range(BBLK // 16)
        )

        @plsc.parallel_loop(0, EMBED, unroll=8)
        def trans(d):
            dd = jnp.full((16,), 0, jnp.int32) + d
            for k in range(BBLK // 16):
                rows = iota16 + (16 * k)
                cols = pcs[k] + dd
                val = plsc.load_gather(bufs[j], [rows, cols])
                bufTs[j][d, pl.ds(16 * k, 16)] = val

    for j in range(NB):
        pltpu.async_copy(tab2.at[idx2v.at[j]], bufs[j], gsems[j])

    def step(t, carry):
        for j in range(NB):
            s = t * NB + j
            pltpu.make_async_copy(tab2.at[idx2v.at[s]], bufs[j], gsems[j]).wait()

            @pl.when(s >= NB)
            def _():
                pltpu.make_async_copy(
                    bufTs[j], out.at[s - NB, :, pl.ds(b0, BBLK)], wsems[j]
                ).wait()

            transpose_group(s, j)
            pltpu.async_copy(bufTs[j], out.at[s, :, pl.ds(b0, BBLK)], wsems[j])

            @pl.when(s + NB < SEQ)
            def _():
                pltpu.async_copy(tab2.at[idx2v.at[s + NB]], bufs[j], gsems[j])
        return carry

    lax.fori_loop(0, SEQ // NB, step, 0)

    for j in range(NB):
        s_last = SEQ - NB + j
        pltpu.make_async_copy(
            bufTs[j], out.at[s_last, :, pl.ds(b0, BBLK)], wsems[j]
        ).wait()


def _pairify(tT, tailp):
    run = pl.kernel(
        _k1_body,
        out_type=jax.ShapeDtypeStruct((PAIR_PAD, 128), jnp.float32),
        mesh=_mesh,
        scratch_types=[pltpu.VMEM((EMBED, 128), jnp.float32) for _ in range(4)]
        + [pltpu.VMEM((40, 128), jnp.float32)]
        + [pltpu.SemaphoreType.DMA] * 4,
        compiler_params=pltpu.CompilerParams(needs_layout_passes=False),
    )
    return run(tT, tailp)


@jax.jit
def _embed(wT, tab2):
    run = pl.kernel(
        _body,
        out_type=jax.ShapeDtypeStruct((SEQ, EMBED, BATCH), jnp.float32),
        mesh=_mesh,
        scratch_types=[
            pltpu.VMEM((SEQ, BBLK), jnp.int32),
            pltpu.VMEM((SEQ, BBLK), jnp.int32),
        ]
        + [pltpu.VMEM((BBLK, 128), jnp.float32) for _ in range(NB)]
        + [pltpu.VMEM((EMBED, BBLK), jnp.float32) for _ in range(NB)]
        + [pltpu.SemaphoreType.DMA] * (2 * NB),
        compiler_params=pltpu.CompilerParams(needs_layout_passes=False),
    )
    return run(wT, tab2)


def kernel(word_batch, table):
    wT = word_batch.astype(jnp.int32).T
    tableT = table.T
    tail = table[TAIL_V : VOCAB2 - 2].reshape(32, 128)
    tailp = jnp.concatenate([tail, jnp.zeros((8, 128), jnp.float32)], axis=0)
    tab2 = _pairify(tableT, tailp)
    outT = _embed(wT, tab2)
    return jnp.transpose(outT, (2, 0, 1))
